# baseline (speedup 1.0000x reference)
import functools

import jax
import jax.numpy as jnp
from jax import lax
from jax.experimental import pallas as pl
from jax.experimental.pallas import tpu as pltpu
from jax.experimental.pallas import tpu_sc as plsc

B = 8
MAX_SEQLEN = 2048
TOTAL_TOK = 8192
D = 1024

NC = 2
NS = 16
NW = NC * NS
TOTAL_ROWS = B * MAX_SEQLEN
RPW = TOTAL_ROWS // NW
WPB = MAX_SEQLEN // RPW
CHUNK = 32
NCHUNK = RPW // CHUNK


def _body(flat_hbm, cu_hbm, z_hbm, out_hbm, cu_v, zbuf):
    wid = lax.axis_index("s") * NC + lax.axis_index("c")
    b = wid // WPB
    j0 = (wid % WPB) * RPW
    row0 = wid * RPW

    pltpu.sync_copy(cu_hbm, cu_v)
    pltpu.sync_copy(z_hbm, zbuf)

    cu_vec = cu_v[...]
    iota = lax.broadcasted_iota(jnp.int32, (16,), 0)
    start = jnp.sum(jnp.where(iota == b, cu_vec, 0))
    end = jnp.sum(jnp.where(iota == b + 1, cu_vec, 0))
    seg_len = jnp.minimum(end - start, MAX_SEQLEN)
    nvalid = jnp.clip(seg_len - j0, 0, RPW)
    kfull = nvalid // CHUNK
    rem = nvalid % CHUNK
    kzero = kfull + (rem > 0).astype(jnp.int32)

    for k in range(NCHUNK):
        dst = out_hbm.at[pl.ds(row0 + k * CHUNK, CHUNK)]

        @pl.when(k < kfull)
        def _():
            pltpu.sync_copy(
                flat_hbm.at[pl.ds(start + j0 + k * CHUNK, CHUNK)], dst)

        @pl.when(k >= kzero)
        def _():
            pltpu.sync_copy(zbuf, dst)

    jb = kfull * CHUNK
    for r in range(CHUNK):
        dst_row = out_hbm.at[row0 + jb + r]

        @pl.when((rem > 0) & (r < rem))
        def _():
            pltpu.sync_copy(flat_hbm.at[start + j0 + jb + r], dst_row)

        @pl.when((rem > 0) & (r >= rem))
        def _():
            pltpu.sync_copy(zbuf.at[0], dst_row)


@jax.jit
def kernel(flat, cu_seqlens):
    cu_pad = jnp.concatenate(
        [cu_seqlens.astype(jnp.int32), jnp.zeros((16 - (B + 1),), jnp.int32)])
    zeros_chunk = jnp.zeros((CHUNK, D), jnp.float32)

    mesh = plsc.VectorSubcoreMesh(core_axis_name="c", subcore_axis_name="s")
    out = pl.kernel(
        _body,
        out_type=jax.ShapeDtypeStruct((TOTAL_ROWS, D), jnp.float32),
        mesh=mesh,
        scratch_types=[
            pltpu.VMEM((16,), jnp.int32),
            pltpu.VMEM((CHUNK, D), jnp.float32),
        ],
        compiler_params=pltpu.CompilerParams(use_tc_tiling_on_sc=False, needs_layout_passes=False),
    )(flat, cu_pad, zeros_chunk)
    return out.reshape(B, MAX_SEQLEN, D)


# speedup vs baseline: 5.3989x; 5.3989x over previous
import jax
import jax.numpy as jnp
from jax import lax
from jax.experimental import pallas as pl
from jax.experimental.pallas import tpu as pltpu
from jax.experimental.pallas import tpu_sc as plsc

B = 8
MAX_SEQLEN = 2048
TOTAL_TOK = 8192
D = 1024

NC = 2
NS = 16
NW = NC * NS
TOTAL_ROWS = B * MAX_SEQLEN
RPW = TOTAL_ROWS // NW
WPB = MAX_SEQLEN // RPW
CHUNK = 32
NCHUNK = RPW // CHUNK
CW = CHUNK * D


def _body(flat_hbm, cu_hbm, z_hbm, out_hbm, cu_v, zbuf, buf0, buf1,
          sg0, sg1, ss0, ss1, sz):
    wid = lax.axis_index("s") * NC + lax.axis_index("c")
    b = wid // WPB
    j0 = (wid % WPB) * RPW
    row0 = wid * RPW

    pltpu.sync_copy(cu_hbm, cu_v)
    pltpu.sync_copy(z_hbm, zbuf)

    cu_vec = cu_v[...]
    iota = lax.broadcasted_iota(jnp.int32, (16,), 0)
    start = jnp.sum(jnp.where(iota == b, cu_vec, 0))
    end = jnp.sum(jnp.where(iota == b + 1, cu_vec, 0))
    seg_len = jnp.minimum(end - start, MAX_SEQLEN)
    nvalid = jnp.clip(seg_len - j0, 0, RPW)
    kfull = nvalid // CHUNK
    rem = nvalid % CHUNK
    kzero = kfull + (rem > 0).astype(jnp.int32)

    bufs = (buf0, buf1)
    sgs = (sg0, sg1)
    sss = (ss0, ss1)

    def out_chunk(k):
        return out_hbm.at[pl.ds((row0 + k * CHUNK) * D, CW)]

    def flat_chunk(k):
        return flat_hbm.at[pl.ds((start + j0 + k * CHUNK) * D, CW)]

    for k in range(NCHUNK):
        @pl.when(k >= kzero)
        def _():
            pltpu.async_copy(zbuf, out_chunk(k), sz)

    for k in range(NCHUNK):
        p = k & 1
        if k >= 2:
            @pl.when(k - 2 < kfull)
            def _():
                pltpu.make_async_copy(bufs[p], out_chunk(k - 2), sss[p]).wait()

        @pl.when(k < kfull)
        def _():
            pltpu.async_copy(flat_chunk(k), bufs[p], sgs[p])

        if k >= 1:
            q = (k - 1) & 1

            @pl.when(k - 1 < kfull)
            def _():
                pltpu.make_async_copy(flat_chunk(k - 1), bufs[q], sgs[q]).wait()
                pltpu.async_copy(bufs[q], out_chunk(k - 1), sss[q])

    @pl.when(NCHUNK - 1 < kfull)
    def _():
        q = (NCHUNK - 1) & 1
        pltpu.make_async_copy(
            flat_chunk(NCHUNK - 1), bufs[q], sgs[q]).wait()
        pltpu.async_copy(bufs[q], out_chunk(NCHUNK - 1), sss[q])

    for k in (NCHUNK - 2, NCHUNK - 1):
        @pl.when(k < kfull)
        def _():
            pltpu.make_async_copy(bufs[k & 1], out_chunk(k), sss[k & 1]).wait()

    jb = kfull * CHUNK

    def out_row(r):
        return out_hbm.at[pl.ds((row0 + jb + r) * D, D)]

    for r in range(CHUNK):
        @pl.when((rem > 0) & (r < rem))
        def _():
            pltpu.async_copy(
                flat_hbm.at[pl.ds((start + j0 + jb + r) * D, D)],
                buf0.at[pl.ds(r * D, D)], sg0)

        @pl.when((rem > 0) & (r >= rem))
        def _():
            pltpu.async_copy(zbuf.at[pl.ds(r * D, D)], out_row(r), sz)

    for r in range(CHUNK):
        @pl.when((rem > 0) & (r < rem))
        def _():
            pltpu.make_async_copy(
                flat_hbm.at[pl.ds((start + j0 + jb + r) * D, D)],
                buf0.at[pl.ds(r * D, D)], sg0).wait()

    for r in range(CHUNK):
        @pl.when((rem > 0) & (r < rem))
        def _():
            pltpu.async_copy(buf0.at[pl.ds(r * D, D)], out_row(r), sz)

    for k in range(NCHUNK):
        @pl.when(k >= kzero)
        def _():
            pltpu.make_async_copy(zbuf, out_chunk(k), sz).wait()

    for r in range(CHUNK):
        @pl.when((rem > 0) & (r >= rem))
        def _():
            pltpu.make_async_copy(
                zbuf.at[pl.ds(r * D, D)], out_row(r), sz).wait()

        @pl.when((rem > 0) & (r < rem))
        def _():
            pltpu.make_async_copy(
                buf0.at[pl.ds(r * D, D)], out_row(r), sz).wait()


@jax.jit
def kernel(flat, cu_seqlens):
    cu_pad = jnp.concatenate(
        [cu_seqlens.astype(jnp.int32), jnp.zeros((16 - (B + 1),), jnp.int32)])
    zeros_chunk = jnp.zeros((CW,), jnp.float32)

    mesh = plsc.VectorSubcoreMesh(core_axis_name="c", subcore_axis_name="s")
    out = pl.kernel(
        _body,
        out_type=jax.ShapeDtypeStruct((TOTAL_ROWS * D,), jnp.float32),
        mesh=mesh,
        scratch_types=[
            pltpu.VMEM((16,), jnp.int32),
            pltpu.VMEM((CW,), jnp.float32),
            pltpu.VMEM((CW,), jnp.float32),
            pltpu.VMEM((CW,), jnp.float32),
            pltpu.SemaphoreType.DMA,
            pltpu.SemaphoreType.DMA,
            pltpu.SemaphoreType.DMA,
            pltpu.SemaphoreType.DMA,
            pltpu.SemaphoreType.DMA,
        ],
        compiler_params=pltpu.CompilerParams(
            use_tc_tiling_on_sc=False, needs_layout_passes=False),
    )(flat.reshape(-1), cu_pad, zeros_chunk)
    return out.reshape(B, MAX_SEQLEN, D)


# speedup vs baseline: 13.9680x; 2.5872x over previous
import jax
import jax.numpy as jnp
from jax import lax
from jax.experimental import pallas as pl
from jax.experimental.pallas import tpu as pltpu
from jax.experimental.pallas import tpu_sc as plsc

B = 8
MAX_SEQLEN = 2048
TOTAL_TOK = 8192
D = 1024

NC = 2
NS = 16
NW = NC * NS
TOTAL_ROWS = B * MAX_SEQLEN
RPW = TOTAL_ROWS // NW
WPB = MAX_SEQLEN // RPW
CHUNK = 32
NCHUNK = RPW // CHUNK
LANES = 16


def _body(flat_hbm, cu_hbm, z_hbm, out_hbm, cu_v, zbuf, buf0, buf1,
          idx_v, idxb_v, sg0, sg1, ss0, ss1, sz):
    wid = lax.axis_index("s") * NC + lax.axis_index("c")
    b = wid // WPB
    j0 = (wid % WPB) * RPW
    row0 = wid * RPW

    pltpu.sync_copy(cu_hbm, cu_v)
    pltpu.sync_copy(z_hbm, zbuf)

    cu_vec = cu_v[...]
    iota = lax.broadcasted_iota(jnp.int32, (LANES,), 0)
    start = jnp.sum(jnp.where(iota == b, cu_vec, 0))
    end = jnp.sum(jnp.where(iota == b + 1, cu_vec, 0))
    seg_len = jnp.minimum(end - start, MAX_SEQLEN)
    nvalid = jnp.clip(seg_len - j0, 0, RPW)
    kfull = nvalid // CHUNK
    rem = nvalid % CHUNK
    kzero = kfull + (rem > 0).astype(jnp.int32)

    base = start + j0
    for k in range(NCHUNK):
        for h in range(0, CHUNK, LANES):
            idx_v[k, pl.ds(h, LANES)] = jnp.minimum(
                base + (k * CHUNK + h) + iota, TOTAL_TOK - 1)
    bbase = base + kfull * CHUNK
    for h in range(0, CHUNK, LANES):
        idxb_v[pl.ds(h, LANES)] = jnp.minimum(
            bbase + h + iota, TOTAL_TOK - 1)

    bufs = (buf0, buf1)
    sgs = (sg0, sg1)
    sss = (ss0, ss1)

    def out_chunk(k):
        return out_hbm.at[pl.ds(pl.multiple_of(row0 + k * CHUNK, CHUNK), CHUNK)]

    for k in range(NCHUNK):
        @pl.when(k >= kzero)
        def _():
            pltpu.async_copy(zbuf, out_chunk(k), sz)

    for k in range(NCHUNK):
        p = k & 1
        if k >= 2:
            @pl.when(k - 2 < kfull)
            def _():
                pltpu.make_async_copy(bufs[p], out_chunk(k - 2), sss[p]).wait()

        @pl.when(k < kfull)
        def _():
            pltpu.async_copy(flat_hbm.at[idx_v.at[k]], bufs[p], sgs[p])

        if k >= 1:
            q = (k - 1) & 1

            @pl.when(k - 1 < kfull)
            def _():
                pltpu.make_async_copy(
                    flat_hbm.at[idx_v.at[k - 1]], bufs[q], sgs[q]).wait()
                pltpu.async_copy(bufs[q], out_chunk(k - 1), sss[q])

    @pl.when(NCHUNK - 1 < kfull)
    def _():
        q = (NCHUNK - 1) & 1
        pltpu.make_async_copy(
            flat_hbm.at[idx_v.at[NCHUNK - 1]], bufs[q], sgs[q]).wait()
        pltpu.async_copy(bufs[q], out_chunk(NCHUNK - 1), sss[q])

    for k in (NCHUNK - 2, NCHUNK - 1):
        @pl.when(k < kfull)
        def _():
            pltpu.make_async_copy(bufs[k & 1], out_chunk(k), sss[k & 1]).wait()

    @pl.when(rem > 0)
    def _():
        pltpu.async_copy(flat_hbm.at[idxb_v], buf0, sg0)
        pltpu.make_async_copy(flat_hbm.at[idxb_v], buf0, sg0).wait()

    @pl.when(rem > 0)
    def _():
        def zero_row(r, carry):
            for i in range(D // LANES):
                buf0[r, pl.ds(i * LANES, LANES)] = jnp.zeros(
                    (LANES,), jnp.float32)
            return carry

        lax.fori_loop(rem, CHUNK, zero_row, 0)

    bchunk_dst = out_hbm.at[
        pl.ds(pl.multiple_of(row0 + kfull * CHUNK, CHUNK), CHUNK)]

    @pl.when(rem > 0)
    def _():
        pltpu.async_copy(buf0, bchunk_dst, sz)

    for k in range(NCHUNK):
        @pl.when(k >= kzero)
        def _():
            pltpu.make_async_copy(zbuf, out_chunk(k), sz).wait()

    @pl.when(rem > 0)
    def _():
        pltpu.make_async_copy(buf0, bchunk_dst, sz).wait()


@jax.jit
def kernel(flat, cu_seqlens):
    cu_pad = jnp.concatenate(
        [cu_seqlens.astype(jnp.int32), jnp.zeros((16 - (B + 1),), jnp.int32)])
    zeros_chunk = jnp.zeros((CHUNK, D), jnp.float32)

    mesh = plsc.VectorSubcoreMesh(core_axis_name="c", subcore_axis_name="s")
    out = pl.kernel(
        _body,
        out_type=jax.ShapeDtypeStruct((TOTAL_ROWS, D), jnp.float32),
        mesh=mesh,
        scratch_types=[
            pltpu.VMEM((LANES,), jnp.int32),
            pltpu.VMEM((CHUNK, D), jnp.float32),
            pltpu.VMEM((CHUNK, D), jnp.float32),
            pltpu.VMEM((CHUNK, D), jnp.float32),
            pltpu.VMEM((NCHUNK, CHUNK), jnp.int32),
            pltpu.VMEM((CHUNK,), jnp.int32),
            pltpu.SemaphoreType.DMA,
            pltpu.SemaphoreType.DMA,
            pltpu.SemaphoreType.DMA,
            pltpu.SemaphoreType.DMA,
            pltpu.SemaphoreType.DMA,
        ],
        compiler_params=pltpu.CompilerParams(needs_layout_passes=False),
    )(flat, cu_pad, zeros_chunk)
    return out.reshape(B, MAX_SEQLEN, D)
